# baseline (speedup 1.0000x reference)
import jax
import jax.numpy as jnp
from jax import lax
from jax.experimental import pallas as pl
from jax.experimental.pallas import tpu as pltpu
from jax.experimental.pallas import tpu_sc as plsc

N_MID = 100000
DIM = 64
B = 4096
SEQ = 50

NW = 32
CHUNK = 128
HIS_PER_W = (B * SEQ) // NW
HIS_CHUNKS = HIS_PER_W // CHUNK
IDXROWS_W = HIS_PER_W // 128
TGT_PER_W = B // NW
NBUF = 3
AHEAD = 2

_mesh = plsc.VectorSubcoreMesh(core_axis_name="c", subcore_axis_name="s")


@pl.kernel(
    out_type=(
        jax.ShapeDtypeStruct((B, 128), jnp.float32),
        jax.ShapeDtypeStruct((B * SEQ, 128), jnp.float32),
    ),
    mesh=_mesh,
    scratch_types=[
        pltpu.VMEM((IDXROWS_W, 128), jnp.int32),
        pltpu.VMEM((IDXROWS_W, 128), jnp.float32),
        pltpu.VMEM((128,), jnp.float32),
        pltpu.VMEM((TGT_PER_W,), jnp.int32),
        pltpu.VMEM((NBUF, CHUNK, DIM), jnp.float32),
        pltpu.VMEM((NBUF, CHUNK, 128), jnp.float32),
        pltpu.VMEM((TGT_PER_W, DIM), jnp.float32),
        pltpu.VMEM((TGT_PER_W, 128), jnp.float32),
        pltpu.SemaphoreType.DMA((NBUF,)),
        pltpu.SemaphoreType.DMA((NBUF,)),
        pltpu.SemaphoreType.DMA,
        pltpu.SemaphoreType.DMA,
    ],
    compiler_params=pltpu.CompilerParams(
        use_tc_tiling_on_sc=False, needs_layout_passes=False
    ),
)
def _lookup(table, his_idx, tgt_idx, mask, out_tgt, out_his,
            idx_v, mask_v, mrow_v, tidx_v, gbuf, obuf, tgbuf, tobuf,
            gsem, ssem, tg, ts):
    wid = lax.axis_index("s") * 2 + lax.axis_index("c")

    pltpu.sync_copy(his_idx.at[pl.ds(wid * IDXROWS_W, IDXROWS_W)], idx_v)
    pltpu.sync_copy(mask.at[pl.ds(wid * IDXROWS_W, IDXROWS_W)], mask_v)
    pltpu.sync_copy(tgt_idx.at[wid], tidx_v)

    tgt_gather = pltpu.make_async_copy(table.at[tidx_v], tgbuf, tg)
    tgt_gather.start()

    his_base = wid * HIS_PER_W

    def gather_start(c, b):
        pltpu.make_async_copy(
            table.at[idx_v.at[c]], gbuf.at[b], gsem.at[b]
        ).start()

    def gather_wait(b):
        pltpu.make_async_copy(
            table.at[idx_v.at[0]], gbuf.at[b], gsem.at[b]
        ).wait()

    def scatter_start(c, b):
        pltpu.make_async_copy(
            obuf.at[b],
            out_his.at[pl.ds(his_base + c * CHUNK, CHUNK)],
            ssem.at[b],
        ).start()

    def scatter_wait(b):
        pltpu.make_async_copy(
            obuf.at[b],
            out_his.at[pl.ds(his_base, CHUNK)],
            ssem.at[b],
        ).wait()

    for b in range(AHEAD):
        gather_start(jnp.int32(b), b)

    tgt_gather.wait()

    def texp(g, carry):
        i0 = g * 4
        for r in range(4):
            for cc in range(4):
                sl = pl.ds(cc * 16, 16)
                tobuf[i0 + r, sl] = tgbuf[i0 + r, sl]
        return carry

    lax.fori_loop(0, TGT_PER_W // 4, texp, 0)
    pltpu.make_async_copy(
        tobuf, out_tgt.at[pl.ds(wid * TGT_PER_W, TGT_PER_W)], ts
    ).start()

    def mul_chunk(c, b):
        for k in range(8):
            sl = pl.ds(k * 16, 16)
            mrow_v[sl] = mask_v[c, sl]

        def grp(g, carry):
            i0 = g * 4
            iv = jnp.broadcast_to(i0, (16,))
            for r in range(4):
                m = plsc.load_gather(mrow_v, [iv + r])
                for cc in range(4):
                    sl = pl.ds(cc * 16, 16)
                    obuf[b, i0 + r, sl] = gbuf[b, i0 + r, sl] * m
            return carry

        lax.fori_loop(0, CHUNK // 4, grp, 0)

    def step(c, b):
        nb = (b + AHEAD) % NBUF
        gather_wait(b)
        mul_chunk(c, b)
        scatter_start(c, b)

        @pl.when(c + AHEAD < HIS_CHUNKS)
        def _():
            @pl.when(c >= 1)
            def _():
                scatter_wait(nb)

            gather_start(c + AHEAD, nb)

    def body(j, carry):
        for b in range(NBUF):
            step(j * NBUF + b, b)
        return carry

    lax.fori_loop(0, HIS_CHUNKS // NBUF, body, 0)
    for b in range(HIS_CHUNKS % NBUF):
        step(jnp.int32((HIS_CHUNKS // NBUF) * NBUF + b), b)

    for b in range(NBUF):
        scatter_wait(b)
    pltpu.make_async_copy(
        tobuf, out_tgt.at[pl.ds(wid * TGT_PER_W, TGT_PER_W)], ts
    ).wait()


_NB = 32


def _his_epilogue(in_ref, out_ref):
    x = in_ref[...]
    out_ref[...] = x[:, :DIM].reshape(_NB, SEQ, DIM)


def _tgt_epilogue(in_ref, out_ref):
    out_ref[...] = in_ref[...][:, :DIM]


def kernel(mid_his_batch_ph, mid_batch_ph, mask, mid_embeddings_var):
    his_idx = mid_his_batch_ph.reshape(B * SEQ // 128, 128)
    tgt_idx = mid_batch_ph.reshape(NW, TGT_PER_W)
    mask2 = mask.reshape(B * SEQ // 128, 128)
    tgt_lines, his_lines = _lookup(
        mid_embeddings_var, his_idx, tgt_idx, mask2
    )
    item_his_eb = pl.pallas_call(
        _his_epilogue,
        grid=(B // _NB,),
        in_specs=[pl.BlockSpec((_NB * SEQ, 128), lambda i: (i, 0))],
        out_specs=pl.BlockSpec((_NB, SEQ, DIM), lambda i: (i, 0, 0)),
        out_shape=jax.ShapeDtypeStruct((B, SEQ, DIM), jnp.float32),
    )(his_lines)
    item_eb = pl.pallas_call(
        _tgt_epilogue,
        grid=(8,),
        in_specs=[pl.BlockSpec((B // 8, 128), lambda i: (i, 0))],
        out_specs=pl.BlockSpec((B // 8, DIM), lambda i: (i, 0)),
        out_shape=jax.ShapeDtypeStruct((B, DIM), jnp.float32),
    )(tgt_lines)
    return item_eb, item_his_eb
